# baseline (speedup 1.0000x reference)
import functools

import jax
import jax.numpy as jnp
from jax import lax
from jax.experimental import pallas as pl
from jax.experimental.pallas import tpu as pltpu
from jax.experimental.pallas import tpu_sc as plsc

N = 10000
E = 320000
D = 128
EPS = 1e-5

NC = 2
NS = 16
NW = NC * NS
CHUNK = 80
EPW = E // NW
CPW = EPW // CHUNK
ECHUNK = 128
ECPW = 80
EPWP = ECPW * ECHUNK
NPAD = 10240
RPT = NPAD // NS
ZROWS = RPT // 5

_mesh = plsc.VectorSubcoreMesh(core_axis_name="c", subcore_axis_name="s")


def _deg_body(dst_hbm, degp_hbm, didx, ones, zbuf, degs, gsem, ssem):
    c = lax.axis_index("c")
    s = lax.axis_index("s")
    wid = c * NS + s

    zero_row = jnp.full((16,), 0.0, jnp.float32)
    one_row = jnp.full((16,), 1.0, jnp.float32)

    def init_ones(r, _):
        for k in range(D // 16):
            ones[r, pl.ds(k * 16, 16)] = one_row
        return 0

    lax.fori_loop(0, CHUNK, init_ones, 0)

    def init_zeros(r, _):
        for k in range(D // 16):
            zbuf[r, pl.ds(k * 16, 16)] = zero_row
        return 0

    lax.fori_loop(0, ZROWS, init_zeros, 0)
    pltpu.async_copy(dst_hbm.at[wid], didx, gsem)
    for t in range(5):
        pltpu.async_copy(zbuf, degs.at[pl.ds(s * RPT + t * ZROWS, ZROWS)],
                         ssem)
    pltpu.make_async_copy(dst_hbm.at[wid], didx, gsem).wait()
    for t in range(5):
        pltpu.make_async_copy(
            zbuf, degs.at[pl.ds(s * RPT + t * ZROWS, ZROWS)], ssem).wait()
    plsc.subcore_barrier()

    def body(jj, _):
        for b in range(5):
            j = jj * 5 + b
            pltpu.async_copy(ones, degs.at[didx.at[j]], ssem, add=True)
        for b in range(5):
            pltpu.make_async_copy(ones, degs.at[didx.at[0]], ssem).wait()
        return 0

    lax.fori_loop(0, CPW // 5, body, 0)
    plsc.subcore_barrier()

    pltpu.sync_copy(degs.at[pl.ds(s * RPT, RPT)],
                    degp_hbm.at[c, pl.ds(s * RPT, RPT)])


@functools.partial(jax.jit)
def _deg_kernel(dst):
    dst3 = dst.reshape(NW, CPW, CHUNK)
    return pl.kernel(
        _deg_body,
        out_type=jax.ShapeDtypeStruct((NC, NPAD, D), jnp.float32),
        mesh=_mesh,
        scratch_types=[
            pltpu.VMEM((CPW, CHUNK), jnp.int32),
            pltpu.VMEM((CHUNK, D), jnp.float32),
            pltpu.VMEM((ZROWS, D), jnp.float32),
            pltpu.VMEM_SHARED((NPAD, D), jnp.float32),
            pltpu.SemaphoreType.DMA,
            pltpu.SemaphoreType.DMA,
        ],
    )(dst3)


def _edge_body(y_hbm, src_hbm, dst_hbm, accp_hbm,
               sbuf0, sbuf1, dbuf0, dbuf1, rows0, rows1, accs,
               gsem0, gsem1, ssem0, ssem1, isem0, isem1):
    c = lax.axis_index("c")
    s = lax.axis_index("s")
    wid = c * NS + s
    sbuf = (sbuf0, sbuf1)
    dbuf = (dbuf0, dbuf1)
    rows = (rows0, rows1)
    gsem = (gsem0, gsem1)
    ssem = (ssem0, ssem1)
    isem = (isem0, isem1)

    zero_row = jnp.full((16,), 0.0, jnp.float32)

    def init_zeros(r, _):
        for k in range(D // 16):
            rows0[r, pl.ds(k * 16, 16)] = zero_row
        return 0

    lax.fori_loop(0, 80, init_zeros, 0)
    for t in range(8):
        pltpu.async_copy(rows0.at[pl.ds(0, 80)],
                         accs.at[pl.ds(s * RPT + t * 80, 80)], ssem0)
    for t in range(8):
        pltpu.make_async_copy(rows0.at[pl.ds(0, 80)],
                              accs.at[pl.ds(s * RPT + t * 80, 80)],
                              ssem0).wait()
    plsc.subcore_barrier()

    def idx_load(j, b, kk):
        pltpu.async_copy(src_hbm.at[pl.ds(wid * EPWP + j * ECHUNK, ECHUNK)], sbuf[b].at[kk], isem[b])
        pltpu.async_copy(dst_hbm.at[pl.ds(wid * EPWP + j * ECHUNK, ECHUNK)], dbuf[b].at[kk], isem[b])

    def idx_drain(b, kk):
        pltpu.make_async_copy(src_hbm.at[pl.ds(0, ECHUNK)], sbuf[b].at[kk],
                              isem[b]).wait()
        pltpu.make_async_copy(dst_hbm.at[pl.ds(0, ECHUNK)], dbuf[b].at[kk],
                              isem[b]).wait()

    def scat_drain(b):
        pltpu.make_async_copy(rows[b], accs.at[dbuf[b].at[0]],
                              ssem[b]).wait()

    idx_load(0, 0, 0)
    idx_load(1, 1, 0)

    def body(jj, _):
        kk = lax.rem(jj, 2)
        for b in range(2):
            j = 2 * jj + b

            @pl.when(j >= 2)
            def _():
                scat_drain(b)

            idx_drain(b, kk)

            @pl.when(j + 2 < ECPW)
            def _():
                idx_load(j + 2, b, 1 - kk)

            pltpu.async_copy(y_hbm.at[sbuf[b].at[kk]], rows[b], gsem[b])
            pltpu.make_async_copy(y_hbm.at[pl.ds(0, ECHUNK)], rows[b],
                                  gsem[b]).wait()
            pltpu.async_copy(rows[b], accs.at[dbuf[b].at[kk]], ssem[b],
                             add=True)
        return 0

    lax.fori_loop(0, ECPW // 2, body, 0)
    scat_drain(0)
    scat_drain(1)
    plsc.subcore_barrier()

    pltpu.sync_copy(accs.at[pl.ds(s * RPT, RPT)],
                    accp_hbm.at[c, pl.ds(s * RPT, RPT)])


@functools.partial(jax.jit)
def _edge_kernel(y, src, dst):
    pad = EPWP - EPW
    src4 = jnp.concatenate(
        [src.reshape(NW, EPW),
         jnp.zeros((NW, pad), jnp.int32)], axis=1).reshape(-1)
    dst4 = jnp.concatenate(
        [dst.reshape(NW, EPW),
         jnp.full((NW, pad), NPAD - 1, jnp.int32)], axis=1).reshape(-1)
    return pl.kernel(
        _edge_body,
        out_type=jax.ShapeDtypeStruct((NC, NPAD, D), jnp.float32),
        mesh=_mesh,
        scratch_types=[
            pltpu.VMEM((2, ECHUNK), jnp.int32),
            pltpu.VMEM((2, ECHUNK), jnp.int32),
            pltpu.VMEM((2, ECHUNK), jnp.int32),
            pltpu.VMEM((2, ECHUNK), jnp.int32),
            pltpu.VMEM((ECHUNK, D), jnp.float32),
            pltpu.VMEM((ECHUNK, D), jnp.float32),
            pltpu.VMEM_SHARED((NPAD, D), jnp.float32),
            pltpu.SemaphoreType.DMA,
            pltpu.SemaphoreType.DMA,
            pltpu.SemaphoreType.DMA,
            pltpu.SemaphoreType.DMA,
            pltpu.SemaphoreType.DMA,
            pltpu.SemaphoreType.DMA,
        ],
    )(y, src4, dst4)


def _dinv_from(degp):
    deg = degp[0, :N, 0] + degp[1, :N, 0]
    return jnp.where(deg > 0, lax.rsqrt(jnp.where(deg > 0, deg, 1.0)),
                     0.0)[:, None]


def _pre_body(x_ref, w_ref, degp_ref, y_ref):
    dinv = _dinv_from(degp_ref[...])
    z = lax.dot_general(x_ref[...], w_ref[...], (((1,), (1,)), ((), ())),
                        preferred_element_type=jnp.float32)
    y_ref[...] = z * dinv


def _pre_kernel(x, w, degp):
    return pl.pallas_call(
        _pre_body,
        out_shape=jax.ShapeDtypeStruct((N, D), jnp.float32),
    )(x, w, degp)


def _bn(h, g, be):
    mu = jnp.mean(h, axis=0, keepdims=True)
    var = jnp.mean((h - mu) ** 2, axis=0, keepdims=True)
    return (h - mu) * lax.rsqrt(var + EPS) * g + be


def _mid_body(accp_ref, degp_ref, b_ref, g_ref, be_ref, w2_ref, y2_ref):
    dinv = _dinv_from(degp_ref[...])
    h = (accp_ref[0, :N] + accp_ref[1, :N]) * dinv + b_ref[...]
    hn = _bn(h, g_ref[...], be_ref[...])
    y2_ref[...] = lax.dot_general(hn, w2_ref[...], (((1,), (1,)), ((), ())),
                                  preferred_element_type=jnp.float32) * dinv


def _mid_kernel(accp, degp, b, g, be, w2):
    return pl.pallas_call(
        _mid_body,
        out_shape=jax.ShapeDtypeStruct((N, D), jnp.float32),
    )(accp, degp, b, g, be, w2)


def _post_body(accp_ref, degp_ref, b_ref, g_ref, be_ref, out_ref):
    dinv = _dinv_from(degp_ref[...])
    h = (accp_ref[0, :N] + accp_ref[1, :N]) * dinv + b_ref[...]
    out_ref[...] = _bn(h, g_ref[...], be_ref[...])


def _post_kernel(accp, degp, b, g, be):
    return pl.pallas_call(
        _post_body,
        out_shape=jax.ShapeDtypeStruct((N, D), jnp.float32),
    )(accp, degp, b, g, be)


def kernel(input_embedding_layer, edges_index, W1, b1, g1, be1, W2, b2, g2,
           be2):
    src = edges_index[0]
    dst = edges_index[1]
    b1r, g1r, be1r = b1[None, :], g1[None, :], be1[None, :]
    b2r, g2r, be2r = b2[None, :], g2[None, :], be2[None, :]

    degp = _deg_kernel(dst)
    y1 = _pre_kernel(input_embedding_layer, W1, degp)
    acc1 = _edge_kernel(y1, src, dst)
    y2 = _mid_kernel(acc1, degp, b1r, g1r, be1r, W2)
    acc2 = _edge_kernel(y2, src, dst)
    return _post_kernel(acc2, degp, b2r, g2r, be2r)


# speedup vs baseline: 2.1830x; 2.1830x over previous
import functools

import jax
import jax.numpy as jnp
from jax import lax
from jax.experimental import pallas as pl
from jax.experimental.pallas import tpu as pltpu
from jax.experimental.pallas import tpu_sc as plsc

N = 10000
E = 320000
D = 128
EPS = 1e-5

NC = 2
NS = 16
NW = NC * NS
CHUNK = 80
EPW = E // NW
CPW = EPW // CHUNK
NPAD = 10240
RPT = NPAD // NS
ZROWS = RPT // 5

_mesh = plsc.VectorSubcoreMesh(core_axis_name="c", subcore_axis_name="s")


def _deg_body(dst_hbm, degp_hbm, didx, ones, zbuf, degs, gsem, ssem):
    c = lax.axis_index("c")
    s = lax.axis_index("s")
    wid = c * NS + s

    zero_row = jnp.full((16,), 0.0, jnp.float32)
    one_row = jnp.full((16,), 1.0, jnp.float32)

    def init_ones(r, _):
        for k in range(D // 16):
            ones[r, pl.ds(k * 16, 16)] = one_row
        return 0

    lax.fori_loop(0, CHUNK, init_ones, 0)

    def init_zeros(r, _):
        for k in range(D // 16):
            zbuf[r, pl.ds(k * 16, 16)] = zero_row
        return 0

    lax.fori_loop(0, ZROWS, init_zeros, 0)
    pltpu.async_copy(dst_hbm.at[wid], didx, gsem)
    for t in range(5):
        pltpu.async_copy(zbuf, degs.at[pl.ds(s * RPT + t * ZROWS, ZROWS)],
                         ssem)
    pltpu.make_async_copy(dst_hbm.at[wid], didx, gsem).wait()
    for t in range(5):
        pltpu.make_async_copy(
            zbuf, degs.at[pl.ds(s * RPT + t * ZROWS, ZROWS)], ssem).wait()
    plsc.subcore_barrier()

    def body(jj, _):
        for b in range(5):
            j = jj * 5 + b
            pltpu.async_copy(ones, degs.at[didx.at[j]], ssem, add=True)
        for b in range(5):
            pltpu.make_async_copy(ones, degs.at[didx.at[0]], ssem).wait()
        return 0

    lax.fori_loop(0, CPW // 5, body, 0)
    plsc.subcore_barrier()

    pltpu.sync_copy(degs.at[pl.ds(s * RPT, RPT)],
                    degp_hbm.at[c, pl.ds(s * RPT, RPT)])


@functools.partial(jax.jit)
def _deg_kernel(dst):
    dst3 = dst.reshape(NW, CPW, CHUNK)
    return pl.kernel(
        _deg_body,
        out_type=jax.ShapeDtypeStruct((NC, NPAD, D), jnp.float32),
        mesh=_mesh,
        scratch_types=[
            pltpu.VMEM((CPW, CHUNK), jnp.int32),
            pltpu.VMEM((CHUNK, D), jnp.float32),
            pltpu.VMEM((ZROWS, D), jnp.float32),
            pltpu.VMEM_SHARED((NPAD, D), jnp.float32),
            pltpu.SemaphoreType.DMA,
            pltpu.SemaphoreType.DMA,
        ],
    )(dst3)


def _edge_body(y_hbm, src_hbm, dst_hbm, accp_hbm,
               sbuf0, sbuf1, dbuf0, dbuf1, rows0, rows1, accs,
               gsem0, gsem1, ssem0, ssem1, isem0, isem1):
    c = lax.axis_index("c")
    s = lax.axis_index("s")
    wid = c * NS + s
    sbuf = (sbuf0, sbuf1)
    dbuf = (dbuf0, dbuf1)
    rows = (rows0, rows1)
    gsem = (gsem0, gsem1)
    ssem = (ssem0, ssem1)
    isem = (isem0, isem1)

    zero_row = jnp.full((16,), 0.0, jnp.float32)

    def init_zeros(r, _):
        for k in range(D // 16):
            rows0[r, pl.ds(k * 16, 16)] = zero_row
        return 0

    lax.fori_loop(0, CHUNK, init_zeros, 0)
    for t in range(8):
        pltpu.async_copy(rows0, accs.at[pl.ds(s * RPT + t * CHUNK, CHUNK)],
                         ssem0)
    for t in range(8):
        pltpu.make_async_copy(
            rows0, accs.at[pl.ds(s * RPT + t * CHUNK, CHUNK)], ssem0).wait()
    plsc.subcore_barrier()

    base = wid * EPW

    def idx_load(j, b, kk):
        pltpu.async_copy(src_hbm.at[pl.ds(base + j * CHUNK, CHUNK)],
                         sbuf[b].at[kk], isem[b])
        pltpu.async_copy(dst_hbm.at[pl.ds(base + j * CHUNK, CHUNK)],
                         dbuf[b].at[kk], isem[b])

    def idx_drain(b, kk):
        pltpu.make_async_copy(src_hbm.at[pl.ds(0, CHUNK)], sbuf[b].at[kk],
                              isem[b]).wait()
        pltpu.make_async_copy(dst_hbm.at[pl.ds(0, CHUNK)], dbuf[b].at[kk],
                              isem[b]).wait()

    def scat_drain(b):
        pltpu.make_async_copy(rows[b], accs.at[dbuf[b].at[0]],
                              ssem[b]).wait()

    idx_load(0, 0, 0)
    idx_load(1, 1, 0)

    def body(jj, _):
        kk = lax.rem(jj, 2)
        for b in range(2):
            j = 2 * jj + b

            @pl.when(j >= 2)
            def _():
                scat_drain(b)

            idx_drain(b, kk)

            @pl.when(j + 2 < CPW)
            def _():
                idx_load(j + 2, b, 1 - kk)

            pltpu.async_copy(y_hbm.at[sbuf[b].at[kk]], rows[b], gsem[b])
            pltpu.make_async_copy(y_hbm.at[pl.ds(0, CHUNK)], rows[b],
                                  gsem[b]).wait()
            pltpu.async_copy(rows[b], accs.at[dbuf[b].at[kk]], ssem[b],
                             add=True)
        return 0

    lax.fori_loop(0, CPW // 2, body, 0)

    kk_l = (CPW // 2) % 2
    scat_drain(0)
    idx_drain(0, kk_l)
    pltpu.async_copy(y_hbm.at[sbuf[0].at[kk_l]], rows0, gsem0)
    pltpu.make_async_copy(y_hbm.at[pl.ds(0, CHUNK)], rows0, gsem0).wait()
    pltpu.async_copy(rows0, accs.at[dbuf[0].at[kk_l]], ssem0, add=True)
    scat_drain(1)
    scat_drain(0)
    plsc.subcore_barrier()

    pltpu.sync_copy(accs.at[pl.ds(s * RPT, RPT)],
                    accp_hbm.at[c, pl.ds(s * RPT, RPT)])


@functools.partial(jax.jit)
def _edge_kernel(y, src, dst):
    return pl.kernel(
        _edge_body,
        out_type=jax.ShapeDtypeStruct((NC, NPAD, D), jnp.float32),
        mesh=_mesh,
        scratch_types=[
            pltpu.VMEM((2, CHUNK), jnp.int32),
            pltpu.VMEM((2, CHUNK), jnp.int32),
            pltpu.VMEM((2, CHUNK), jnp.int32),
            pltpu.VMEM((2, CHUNK), jnp.int32),
            pltpu.VMEM((CHUNK, D), jnp.float32),
            pltpu.VMEM((CHUNK, D), jnp.float32),
            pltpu.VMEM_SHARED((NPAD, D), jnp.float32),
            pltpu.SemaphoreType.DMA,
            pltpu.SemaphoreType.DMA,
            pltpu.SemaphoreType.DMA,
            pltpu.SemaphoreType.DMA,
            pltpu.SemaphoreType.DMA,
            pltpu.SemaphoreType.DMA,
        ],
    )(y, src, dst)


def _dinv_from(degp):
    deg = degp[0, :N, 0] + degp[1, :N, 0]
    return jnp.where(deg > 0, lax.rsqrt(jnp.where(deg > 0, deg, 1.0)),
                     0.0)[:, None]


def _pre_body(x_ref, w_ref, degp_ref, y_ref):
    dinv = _dinv_from(degp_ref[...])
    z = lax.dot_general(x_ref[...], w_ref[...], (((1,), (1,)), ((), ())),
                        preferred_element_type=jnp.float32)
    y_ref[...] = z * dinv


def _pre_kernel(x, w, degp):
    return pl.pallas_call(
        _pre_body,
        out_shape=jax.ShapeDtypeStruct((N, D), jnp.float32),
    )(x, w, degp)


def _bn(h, g, be):
    mu = jnp.mean(h, axis=0, keepdims=True)
    var = jnp.mean((h - mu) ** 2, axis=0, keepdims=True)
    return (h - mu) * lax.rsqrt(var + EPS) * g + be


def _mid_body(accp_ref, degp_ref, b_ref, g_ref, be_ref, w2_ref, y2_ref):
    dinv = _dinv_from(degp_ref[...])
    h = (accp_ref[0, :N] + accp_ref[1, :N]) * dinv + b_ref[...]
    hn = _bn(h, g_ref[...], be_ref[...])
    y2_ref[...] = lax.dot_general(hn, w2_ref[...], (((1,), (1,)), ((), ())),
                                  preferred_element_type=jnp.float32) * dinv


def _mid_kernel(accp, degp, b, g, be, w2):
    return pl.pallas_call(
        _mid_body,
        out_shape=jax.ShapeDtypeStruct((N, D), jnp.float32),
    )(accp, degp, b, g, be, w2)


def _post_body(accp_ref, degp_ref, b_ref, g_ref, be_ref, out_ref):
    dinv = _dinv_from(degp_ref[...])
    h = (accp_ref[0, :N] + accp_ref[1, :N]) * dinv + b_ref[...]
    out_ref[...] = _bn(h, g_ref[...], be_ref[...])


def _post_kernel(accp, degp, b, g, be):
    return pl.pallas_call(
        _post_body,
        out_shape=jax.ShapeDtypeStruct((N, D), jnp.float32),
    )(accp, degp, b, g, be)


def kernel(input_embedding_layer, edges_index, W1, b1, g1, be1, W2, b2, g2,
           be2):
    src = edges_index[0]
    dst = edges_index[1]
    b1r, g1r, be1r = b1[None, :], g1[None, :], be1[None, :]
    b2r, g2r, be2r = b2[None, :], g2[None, :], be2[None, :]

    degp = _deg_kernel(dst)
    y1 = _pre_kernel(input_embedding_layer, W1, degp)
    acc1 = _edge_kernel(y1, src, dst)
    y2 = _mid_kernel(acc1, degp, b1r, g1r, be1r, W2)
    acc2 = _edge_kernel(y2, src, dst)
    return _post_kernel(acc2, degp, b2r, g2r, be2r)
